# baseline (speedup 1.0000x reference)
import jax
import jax.numpy as jnp
from jax import lax
from jax.experimental import pallas as pl
from jax.experimental.pallas import tpu as pltpu

EPS = 1e-5


def _make_kernel(nk, ch):
    def _norm_kernel(*refs):
        m_ref, g1_ref, bt_ref, o_ref = refs[nk], refs[nk + 1], refs[nk + 2], refs[nk + 3]
        m = m_ref[...]
        num = jnp.sum(m, axis=-1, keepdims=True)
        inv = 1.0 / (num + EPS)
        fg = m > 0.0
        for i in range(nk):
            x = refs[i][...]
            r = m * x
            s1 = jnp.sum(r, axis=-1, keepdims=True)
            s2 = jnp.sum(r * r, axis=-1, keepdims=True)
            mu = s1 * inv
            var = jnp.maximum(s2 * inv - mu * mu, 0.0)
            a = lax.rsqrt(var + EPS) * g1_ref[:, i * ch:(i + 1) * ch, :]
            b = bt_ref[:, i * ch:(i + 1) * ch, :] - mu * a
            o_ref[:, i * ch:(i + 1) * ch, :] = jnp.where(fg, x * a + b, x)
    return _norm_kernel


def kernel(x, mask, gamma, beta):
    N, C, H, W = x.shape
    mh, mw = mask.shape[2], mask.shape[3]
    fh, fw = H // mh, W // mw
    HW = H * W

    m = jnp.broadcast_to(
        mask.reshape(N, 1, mh, 1, mw, 1), (N, 1, mh, fh, mw, fw)
    ).reshape(N, 1, HW).astype(jnp.float32)

    x_f = x.reshape(N, C, HW)
    g1 = (1.0 + gamma).astype(jnp.float32).reshape(1, C, 1)
    bt = beta.astype(jnp.float32).reshape(1, C, 1)

    B = 1
    K = 4 if C % 4 == 0 else 1
    Ch = C // K
    grid = (N // B,)

    out = pl.pallas_call(
        _make_kernel(K, Ch),
        out_shape=jax.ShapeDtypeStruct((N, C, HW), x.dtype),
        grid=grid,
        in_specs=(
            [pl.BlockSpec((B, Ch, HW), lambda n, i=i: (n, i, 0))
             for i in range(K)]
            + [pl.BlockSpec((B, 1, HW), lambda n: (n, 0, 0)),
               pl.BlockSpec((1, C, 1), lambda n: (0, 0, 0)),
               pl.BlockSpec((1, C, 1), lambda n: (0, 0, 0))]
        ),
        out_specs=pl.BlockSpec((B, C, HW), lambda n: (n, 0, 0)),
        compiler_params=pltpu.CompilerParams(
            dimension_semantics=("parallel",),
            vmem_limit_bytes=64 * 1024 * 1024,
        ),
    )(*([x_f] * K + [m, g1, bt]))
    return out.reshape(N, C, H, W)


# speedup vs baseline: 1.0061x; 1.0061x over previous
import jax
import jax.numpy as jnp
from jax import lax
from jax.experimental import pallas as pl
from jax.experimental.pallas import tpu as pltpu
from jax._src.pallas.mosaic.primitives import make_async_copy as _make_async_copy

EPS = 1e-5


def _make_kernel(nk, ch, nb):
    def _norm_kernel(*refs):
        xs = refs[:nk]
        m_hbm, g1_hbm, bt_hbm, o_ref = refs[nk:nk + 4]
        m_s, g1_s, bt_s, sem_m, sem_g, sem_b = refs[nk + 4:nk + 10]
        n0 = pl.program_id(0)

        @pl.when(n0 == 0)
        def _():
            cm = _make_async_copy(m_hbm, m_s, sem_m)
            cg = _make_async_copy(g1_hbm, g1_s, sem_g)
            cb = _make_async_copy(bt_hbm, bt_s, sem_b)
            cm.start()
            cg.start()
            cb.start()
            cm.wait()
            cg.wait()
            cb.wait()

        m = m_s[pl.ds(n0 * nb, nb), :, :]
        num = jnp.sum(m, axis=-1, keepdims=True)
        inv = 1.0 / (num + EPS)
        fg = m > 0.0
        for i in range(nk):
            x = xs[i][...]
            r = m * x
            s1 = jnp.sum(r, axis=-1, keepdims=True)
            s2 = jnp.sum(r * r, axis=-1, keepdims=True)
            mu = s1 * inv
            var = jnp.maximum(s2 * inv - mu * mu, 0.0)
            a = lax.rsqrt(var + EPS) * g1_s[:, i * ch:(i + 1) * ch, :]
            b = bt_s[:, i * ch:(i + 1) * ch, :] - mu * a
            o_ref[:, i * ch:(i + 1) * ch, :] = jnp.where(fg, x * a + b, x)
    return _norm_kernel


def kernel(x, mask, gamma, beta):
    N, C, H, W = x.shape
    mh, mw = mask.shape[2], mask.shape[3]
    fh, fw = H // mh, W // mw
    HW = H * W

    m = jnp.broadcast_to(
        mask.reshape(N, 1, mh, 1, mw, 1), (N, 1, mh, fh, mw, fw)
    ).reshape(N, 1, HW).astype(jnp.float32)

    x_f = x.reshape(N, C, HW)
    g1 = (1.0 + gamma).astype(jnp.float32).reshape(1, C, 1)
    bt = beta.astype(jnp.float32).reshape(1, C, 1)

    B = 2 if N % 2 == 0 else 1
    K = 4 if C % 4 == 0 else 1
    Ch = C // K
    grid = (N // B,)

    out = pl.pallas_call(
        _make_kernel(K, Ch, B),
        out_shape=jax.ShapeDtypeStruct((N, C, HW), x.dtype),
        grid=grid,
        in_specs=(
            [pl.BlockSpec((B, Ch, HW), lambda n, i=i: (n, i, 0))
             for i in range(K)]
            + [pl.BlockSpec(memory_space=pl.ANY),
               pl.BlockSpec(memory_space=pl.ANY),
               pl.BlockSpec(memory_space=pl.ANY)]
        ),
        out_specs=pl.BlockSpec((B, C, HW), lambda n: (n, 0, 0)),
        scratch_shapes=[
            pltpu.VMEM((N, 1, HW), jnp.float32),
            pltpu.VMEM((1, C, 1), jnp.float32),
            pltpu.VMEM((1, C, 1), jnp.float32),
            pltpu.SemaphoreType.DMA,
            pltpu.SemaphoreType.DMA,
            pltpu.SemaphoreType.DMA,
        ],
        compiler_params=pltpu.CompilerParams(
            dimension_semantics=("arbitrary",),
            vmem_limit_bytes=64 * 1024 * 1024,
        ),
    )(*([x_f] * K + [m, g1, bt]))
    return out.reshape(N, C, H, W)
